# baseline (speedup 1.0000x reference)
import functools

import jax
import jax.numpy as jnp
import numpy as np
from jax import lax
from jax.experimental import pallas as pl
from jax.experimental.pallas import tpu as pltpu
from jax.experimental.pallas import tpu_sc as plsc

N = 2048
P = 16384
D = 1024
HID = 150
HP = 256
B = 512
NC, NS = 2, 16
NW = NC * NS
H = P // 2
NBH = H // B
PPW = H // NW
C = 32
NCH = PPW // C
BF16 = jnp.bfloat16
I32 = jnp.int32
DW = D // 2
HW = HP // 2
F32 = jnp.float32


def _rnd(x):
    b = lax.bitcast_convert_type(x, jnp.int32)
    return b + 0x7FFF + ((b >> 16) & 1)


def _pack(x, half):
    xl = x[:, :half]
    xh = x[:, half:]
    return (lax.shift_right_logical(_rnd(xl), 16)
            | (_rnd(xh) & jnp.int32(-65536)))


def _prep_kernel(g_ref, w1a_ref, w1b_ref, ms_ref, b1_ref, e48_ref, w1d_ref,
                 gb_ref, gm_ref, ga_ref, phi_ref):
    g = g_ref[...]
    col = lax.broadcasted_iota(jnp.int32, (1, HP), 1)
    ms = ms_ref[...]
    ms_hi = ms.astype(jnp.bfloat16).astype(F32)
    ms_lo = ms - ms_hi
    sel = lambda c: (col == c).astype(F32)
    gb_ref[...] = _pack(g, DW)
    gm = (jnp.dot(g, w1a_ref[...], preferred_element_type=F32)
          + b1_ref[...] + ms_hi * sel(150) + ms_lo * sel(151))
    ga = (jnp.dot(g, w1b_ref[...], preferred_element_type=F32)
          + ms_hi * sel(152) + ms_lo * sel(153))
    gm_ref[...] = _pack(gm, HW)
    ga_ref[...] = _pack(ga, HW)
    phi_ref[...] = jnp.dot(e48_ref[...], w1d_ref[...], preferred_element_type=F32)


def _sc_body(half, g_hbm, gm_hbm, ga_hbm, mid_hbm, aid_hbm,
             xi_hbm, xj_hbm, am_hbm, aa_hbm,
             midx, aidx, gi, gj, gm, ga,
             s_gi, s_gj, s_gm, s_ga, s_wi, s_wj, s_wm, s_wa):
    wid = lax.axis_index("s") * NC + lax.axis_index("c")
    base = wid * PPW
    src = half * H + base
    pltpu.sync_copy(mid_hbm.at[pl.ds(src, PPW)], midx)
    pltpu.sync_copy(aid_hbm.at[pl.ds(src, PPW)], aidx)

    def gather_descs(k):
        off = (k % 3) * C
        i_idx = midx.at[pl.ds(k * C, C)]
        j_idx = aidx.at[pl.ds(k * C, C)]
        return (
            (g_hbm.at[i_idx], gi.at[pl.ds(off, C)], s_gi),
            (g_hbm.at[j_idx], gj.at[pl.ds(off, C)], s_gj),
            (gm_hbm.at[i_idx], gm.at[pl.ds(off, C)], s_gm),
            (ga_hbm.at[j_idx], ga.at[pl.ds(off, C)], s_ga),
        )

    def write_descs(k):
        off = (k % 3) * C
        row = base + k * C
        return (
            (gi.at[pl.ds(off, C)], xi_hbm.at[pl.ds(row, C)], s_wi),
            (gj.at[pl.ds(off, C)], xj_hbm.at[pl.ds(row, C)], s_wj),
            (gm.at[pl.ds(off, C)], am_hbm.at[pl.ds(row, C)], s_wm),
            (ga.at[pl.ds(off, C)], aa_hbm.at[pl.ds(row, C)], s_wa),
        )

    def issue(descs):
        for s, d, sem in descs:
            pltpu.async_copy(s, d, sem)

    def wait(descs):
        for s, d, sem in descs:
            pltpu.make_async_copy(s, d, sem).wait()

    issue(gather_descs(0))

    def chunk(k, _):
        @pl.when(k + 1 < NCH)
        def _():
            @pl.when(k >= 2)
            def _():
                wait(write_descs(k - 2))
            issue(gather_descs(k + 1))

        wait(gather_descs(k))
        issue(write_descs(k))
        return 0

    lax.fori_loop(0, NCH, chunk, 0)
    wait(write_descs(NCH - 3))
    wait(write_descs(NCH - 2))
    wait(write_descs(NCH - 1))


def _unpk(w):
    lo = lax.bitcast_convert_type(w << 16, F32)
    hi = lax.bitcast_convert_type(w & jnp.int32(-65536), F32)
    return lo, hi


def _mlp_kernel(xi_ref, xj_ref, am_ref, aa_ref, mid_ref, did_ref, gid_ref,
                sid_ref, w1c_ref, phi_ref, w2_ref, b2_ref,
                w3_ref, b3_ref, mh1_ref, msv_ref, e_ref, den_ref):
    i = pl.program_id(0)
    xie, xio = _unpk(xi_ref[...])
    xje, xjo = _unpk(xj_ref[...])
    pe = (xie * xje).astype(BF16)
    po = (xio * xjo).astype(BF16)
    p = jnp.concatenate([pe, po], axis=1)
    ame, amo = _unpk(am_ref[...])
    aae, aao = _unpk(aa_ref[...])
    aff = jnp.concatenate([ame + aae, amo + aao], axis=1)

    d = did_ref[...]
    gd = gid_ref[...]
    sp = sid_ref[...]
    i48 = lax.broadcasted_iota(jnp.int32, (B, 48), 1)
    oh = ((i48 == d) | (i48 == gd + 16) | (i48 == sp + 32)).astype(F32)

    h1 = jnp.dot(p, w1c_ref[...], preferred_element_type=F32)
    h1 = h1 + jnp.dot(oh, phi_ref[...], preferred_element_type=F32)
    h1 = jnp.maximum(h1 + aff * mh1_ref[...], 0.0)
    h2 = jnp.maximum(jnp.dot(h1, w2_ref[...], preferred_element_type=F32)
                     + b2_ref[...], 0.0)
    sij = jnp.sum(h2 * w3_ref[...], axis=1, keepdims=True)
    sv = jnp.sum(aff * msv_ref[...], axis=1, keepdims=True)
    coref = sij + b3_ref[...] + sv
    e = jnp.exp(coref)
    e_ref[...] = e

    mid = mid_ref[...]
    iN = lax.broadcasted_iota(jnp.int32, (B, N), 1)
    mask = (iN == mid).astype(F32)

    @pl.when(i == 0)
    def _():
        den_ref[...] = jnp.ones_like(den_ref)

    den_ref[...] += lax.dot_general(e, mask, (((0,), (0,)), ((), ())),
                                    preferred_element_type=F32)


def _probs_kernel(e_ref, mid_ref, den0_ref, den1_ref, p_ref, eps_ref):
    i = pl.program_id(0)
    den = den0_ref[...] + den1_ref[...] - 1.0
    e = e_ref[...]
    mid = mid_ref[...]
    iN = lax.broadcasted_iota(jnp.int32, (B, N), 1)
    mask = (iN == mid).astype(F32)
    dsel = jnp.sum(mask * den, axis=1, keepdims=True)
    p_ref[...] = e / dsel

    @pl.when(i == 0)
    def _():
        eps_ref[...] = 1.0 / den


def kernel(g_i, mention_scores, mention_ids, antecedent_ids, distance_ids,
           genre_ids, speaker_ids, W_dist, W_genre, W_speaker,
           W1, b1, W2, b2, W3, b3):
    pad = HP - HID
    w1aT = jnp.pad(W1[:, :D].T, ((0, 0), (0, pad)))
    w1bT = jnp.pad(W1[:, D:2 * D].T, ((0, 0), (0, pad)))
    w1cT = jnp.pad(W1[:, 2 * D:3 * D].T, ((0, 0), (0, pad)))
    w1dT = jnp.pad(W1[:, 3 * D:].T, ((0, 0), (0, pad)))
    b1p = jnp.pad(b1, (0, pad)).reshape(1, HP)
    b2p = jnp.pad(b2, (0, pad)).reshape(1, HP)
    w3row = jnp.pad(W3[0], (0, pad)).reshape(1, HP)
    b3a = b3.reshape(1, 1)
    e48 = jnp.zeros((48, 60), F32)
    e48 = e48.at[0:9, 0:20].set(W_dist)
    e48 = e48.at[16:24, 20:40].set(W_genre)
    e48 = e48.at[32:35, 40:60].set(W_speaker)

    g3, gm3, ga3, phiT = pl.pallas_call(
        _prep_kernel,
        out_shape=[
            jax.ShapeDtypeStruct((N, DW), I32),
            jax.ShapeDtypeStruct((N, HW), I32),
            jax.ShapeDtypeStruct((N, HW), I32),
            jax.ShapeDtypeStruct((48, HP), F32),
        ],
    )(g_i, w1aT, w1bT, mention_scores, b1p, e48, w1dT)

    sc = [pl.kernel(
        functools.partial(_sc_body, h),
        out_type=[
            jax.ShapeDtypeStruct((H, DW), I32),
            jax.ShapeDtypeStruct((H, DW), I32),
            jax.ShapeDtypeStruct((H, HW), I32),
            jax.ShapeDtypeStruct((H, HW), I32),
        ],
        mesh=plsc.VectorSubcoreMesh(core_axis_name="c", subcore_axis_name="s",
                                    num_cores=NC, num_subcores=NS),
        scratch_types=[
            pltpu.VMEM((PPW,), jnp.int32),
            pltpu.VMEM((PPW,), jnp.int32),
            pltpu.VMEM((3 * C, DW), I32),
            pltpu.VMEM((3 * C, DW), I32),
            pltpu.VMEM((3 * C, HW), I32),
            pltpu.VMEM((3 * C, HW), I32),
        ] + [pltpu.SemaphoreType.DMA] * 8,
    ) for h in range(2)]
    mid2 = mention_ids.reshape(P, 1)
    did2 = distance_ids.reshape(P, 1)
    gid2 = genre_ids.reshape(P, 1)
    sid2 = speaker_ids.reshape(P, 1)
    w2T = jnp.pad(W2.T, ((0, pad), (0, pad)))

    row = lambda i: (i, 0)
    full2 = lambda i: (0, 0)

    def make_mlp(h):
        hrow = lambda i: (i + h * NBH, 0)
        return pl.pallas_call(
            _mlp_kernel,
            grid=(NBH,),
            in_specs=[
                pl.BlockSpec((B, DW), row),
                pl.BlockSpec((B, DW), row),
                pl.BlockSpec((B, HW), row),
                pl.BlockSpec((B, HW), row),
                pl.BlockSpec((B, 1), hrow),
                pl.BlockSpec((B, 1), hrow),
                pl.BlockSpec((B, 1), hrow),
                pl.BlockSpec((B, 1), hrow),
                pl.BlockSpec((D, HP), full2),
                pl.BlockSpec((48, HP), full2),
                pl.BlockSpec((HP, HP), full2),
                pl.BlockSpec((1, HP), full2),
                pl.BlockSpec((1, HP), full2),
                pl.BlockSpec((1, 1), full2),
                pl.BlockSpec((1, HP), full2),
                pl.BlockSpec((1, HP), full2),
            ],
            out_specs=[
                pl.BlockSpec((B, 1), row),
                pl.BlockSpec((1, N), full2),
            ],
            out_shape=[
                jax.ShapeDtypeStruct((H, 1), F32),
                jax.ShapeDtypeStruct((1, N), F32),
            ],
        )

    w1c_hi = w1cT.astype(BF16)
    col = np.arange(HP)
    mh1 = jnp.asarray((col < HID).astype(np.float32)).reshape(1, HP)
    msv = jnp.asarray(((col >= HID) & (col <= HID + 3))
                      .astype(np.float32)).reshape(1, HP)

    es, dens = [], []
    for h in range(2):
        xi, xj, am, aa = sc[h](g3, gm3, ga3, mention_ids, antecedent_ids)
        e2, den = make_mlp(h)(xi, xj, am, aa, mid2, did2, gid2, sid2,
                              w1c_hi, phiT, w2T, b2p, w3row, b3a, mh1, msv)
        es.append(e2)
        dens.append(den)

    ps = []
    for h in range(2):
        hrow = lambda i, h=h: (i + h * NBH, 0)
        p2, eps = pl.pallas_call(
            _probs_kernel,
            grid=(NBH,),
            in_specs=[
                pl.BlockSpec((B, 1), row),
                pl.BlockSpec((B, 1), hrow),
                pl.BlockSpec((1, N), full2),
                pl.BlockSpec((1, N), full2),
            ],
            out_specs=[
                pl.BlockSpec((B, 1), row),
                pl.BlockSpec((1, N), full2),
            ],
            out_shape=[
                jax.ShapeDtypeStruct((H, 1), F32),
                jax.ShapeDtypeStruct((1, N), F32),
            ],
        )(es[h], mid2, dens[0], dens[1])
        ps.append(p2)

    return jnp.concatenate([ps[0].reshape(H), ps[1].reshape(H),
                            eps.reshape(N)])


# speedup vs baseline: 1.1499x; 1.1499x over previous
import functools

import jax
import jax.numpy as jnp
import numpy as np
from jax import lax
from jax.experimental import pallas as pl
from jax.experimental.pallas import tpu as pltpu
from jax.experimental.pallas import tpu_sc as plsc

N = 2048
P = 16384
D = 1024
HID = 150
HP = 256
B = 512
NC, NS = 2, 16
NW = NC * NS
H = P // 2
NBH = H // B
PPW = H // NW
C = 32
NCH = PPW // C
BF16 = jnp.bfloat16
I32 = jnp.int32
DW = D // 2
HW = HP // 2
F32 = jnp.float32


def _rnd(x):
    b = lax.bitcast_convert_type(x, jnp.int32)
    return b + 0x7FFF + ((b >> 16) & 1)


def _pack(x, half):
    xl = x[:, :half]
    xh = x[:, half:]
    return (lax.shift_right_logical(_rnd(xl), 16)
            | (_rnd(xh) & jnp.int32(-65536)))


def _prep_kernel(g_ref, w1a_ref, w1b_ref, ms_ref, b1_ref, e48_ref, w1d_ref,
                 gb_ref, gm_ref, ga_ref, phi_ref):
    g = g_ref[...]
    col = lax.broadcasted_iota(jnp.int32, (1, HP), 1)
    ms = ms_ref[...]
    ms_hi = ms.astype(jnp.bfloat16).astype(F32)
    ms_lo = ms - ms_hi
    sel = lambda c: (col == c).astype(F32)
    gb_ref[...] = _pack(g, DW)
    gm = (jnp.dot(g, w1a_ref[...], preferred_element_type=F32)
          + b1_ref[...] + ms_hi * sel(150) + ms_lo * sel(151))
    ga = (jnp.dot(g, w1b_ref[...], preferred_element_type=F32)
          + ms_hi * sel(152) + ms_lo * sel(153))
    gm_ref[...] = _pack(gm, HW)
    ga_ref[...] = _pack(ga, HW)
    phi_ref[...] = jnp.dot(e48_ref[...], w1d_ref[...], preferred_element_type=F32)


def _sc_body(half, g_hbm, gm_hbm, ga_hbm, mid_hbm, aid_hbm,
             xi_hbm, xj_hbm, am_hbm, aa_hbm,
             midx, aidx, gi, gj, gm, ga,
             s_gi, s_gj, s_gm, s_ga, s_wi, s_wj, s_wm, s_wa):
    wid = lax.axis_index("s") * NC + lax.axis_index("c")
    base = wid * PPW
    src = half * H + base
    pltpu.sync_copy(mid_hbm.at[pl.ds(src, PPW)], midx)
    pltpu.sync_copy(aid_hbm.at[pl.ds(src, PPW)], aidx)

    def gather_descs(k):
        off = (k % 3) * C
        i_idx = midx.at[pl.ds(k * C, C)]
        j_idx = aidx.at[pl.ds(k * C, C)]
        return (
            (g_hbm.at[i_idx], gi.at[pl.ds(off, C)], s_gi),
            (g_hbm.at[j_idx], gj.at[pl.ds(off, C)], s_gj),
            (gm_hbm.at[i_idx], gm.at[pl.ds(off, C)], s_gm),
            (ga_hbm.at[j_idx], ga.at[pl.ds(off, C)], s_ga),
        )

    def write_descs(k):
        off = (k % 3) * C
        row = base + k * C
        return (
            (gi.at[pl.ds(off, C)], xi_hbm.at[pl.ds(row, C)], s_wi),
            (gj.at[pl.ds(off, C)], xj_hbm.at[pl.ds(row, C)], s_wj),
            (gm.at[pl.ds(off, C)], am_hbm.at[pl.ds(row, C)], s_wm),
            (ga.at[pl.ds(off, C)], aa_hbm.at[pl.ds(row, C)], s_wa),
        )

    def issue(descs):
        for s, d, sem in descs:
            pltpu.async_copy(s, d, sem)

    def wait(descs):
        for s, d, sem in descs:
            pltpu.make_async_copy(s, d, sem).wait()

    issue(gather_descs(0))

    def chunk(k, _):
        @pl.when(k + 1 < NCH)
        def _():
            @pl.when(k >= 2)
            def _():
                wait(write_descs(k - 2))
            issue(gather_descs(k + 1))

        wait(gather_descs(k))
        issue(write_descs(k))
        return 0

    lax.fori_loop(0, NCH, chunk, 0)
    wait(write_descs(NCH - 3))
    wait(write_descs(NCH - 2))
    wait(write_descs(NCH - 1))


def _unpk(w):
    lo = lax.bitcast_convert_type(w << 16, F32)
    hi = lax.bitcast_convert_type(w & jnp.int32(-65536), F32)
    return lo, hi


def _mlp_kernel(xi_ref, xj_ref, am_ref, aa_ref, mid_ref, did_ref, gid_ref,
                sid_ref, w1c_ref, phi_ref, w2_ref, b2_ref,
                w3_ref, b3_ref, mh1_ref, msv_ref, e_ref, den_ref):
    i = pl.program_id(0)
    xie, xio = _unpk(xi_ref[...])
    xje, xjo = _unpk(xj_ref[...])
    pe = (xie * xje).astype(BF16)
    po = (xio * xjo).astype(BF16)
    p = jnp.concatenate([pe, po], axis=1)
    ame, amo = _unpk(am_ref[...])
    aae, aao = _unpk(aa_ref[...])
    aff = jnp.concatenate([ame + aae, amo + aao], axis=1)

    d = did_ref[0, 0, :].reshape(1, B)
    gd = gid_ref[0, 0, :].reshape(1, B)
    sp = sid_ref[0, 0, :].reshape(1, B)
    i48 = lax.broadcasted_iota(jnp.int32, (48, B), 0)
    ohT = ((i48 == d) | (i48 == gd + 16) | (i48 == sp + 32)).astype(F32)

    h1 = jnp.dot(p, w1c_ref[...], preferred_element_type=F32)
    h1 = h1 + lax.dot_general(ohT, phi_ref[...], (((0,), (0,)), ((), ())),
                              preferred_element_type=F32)
    h1 = jnp.maximum(h1 + aff * mh1_ref[...], 0.0)
    h2 = jnp.maximum(jnp.dot(h1, w2_ref[...], preferred_element_type=F32)
                     + b2_ref[...], 0.0)
    sij = lax.dot_general(w3_ref[...], h2, (((1,), (1,)), ((), ())),
                          preferred_element_type=F32)
    sv = lax.dot_general(msv_ref[...], aff, (((1,), (1,)), ((), ())),
                         preferred_element_type=F32)
    e = jnp.exp(sij + sv + b3_ref[...])
    e_ref[...] = e.reshape(1, 1, B)

    mid = mid_ref[0, 0, :].reshape(1, B)
    iN = lax.broadcasted_iota(jnp.int32, (N, B), 0)
    maskT = (iN == mid).astype(F32)

    @pl.when(i == 0)
    def _():
        den_ref[...] = jnp.ones_like(den_ref)

    den_ref[...] += lax.dot_general(e, maskT, (((1,), (1,)), ((), ())),
                                    preferred_element_type=F32)


def _probs_kernel(e_ref, mid_ref, den0_ref, den1_ref, p_ref, eps_ref):
    i = pl.program_id(0)
    den = den0_ref[...] + den1_ref[...] - 1.0
    e = e_ref[0, 0, :].reshape(1, B)
    mid = mid_ref[0, 0, :].reshape(1, B)
    iN = lax.broadcasted_iota(jnp.int32, (N, B), 0)
    maskT = (iN == mid).astype(F32)
    dsel = lax.dot_general(den, maskT, (((1,), (0,)), ((), ())),
                           preferred_element_type=F32)
    p_ref[...] = (e / dsel).reshape(1, 1, B)

    @pl.when(i == 0)
    def _():
        eps_ref[...] = 1.0 / den


def kernel(g_i, mention_scores, mention_ids, antecedent_ids, distance_ids,
           genre_ids, speaker_ids, W_dist, W_genre, W_speaker,
           W1, b1, W2, b2, W3, b3):
    pad = HP - HID
    w1aT = jnp.pad(W1[:, :D].T, ((0, 0), (0, pad)))
    w1bT = jnp.pad(W1[:, D:2 * D].T, ((0, 0), (0, pad)))
    w1cT = jnp.pad(W1[:, 2 * D:3 * D].T, ((0, 0), (0, pad)))
    w1dT = jnp.pad(W1[:, 3 * D:].T, ((0, 0), (0, pad)))
    b1p = jnp.pad(b1, (0, pad)).reshape(1, HP)
    b2p = jnp.pad(b2, (0, pad)).reshape(1, HP)
    w3row = jnp.pad(W3[0], (0, pad)).reshape(1, HP)
    b3a = b3.reshape(1, 1)
    e48 = jnp.zeros((48, 60), F32)
    e48 = e48.at[0:9, 0:20].set(W_dist)
    e48 = e48.at[16:24, 20:40].set(W_genre)
    e48 = e48.at[32:35, 40:60].set(W_speaker)

    g3, gm3, ga3, phiT = pl.pallas_call(
        _prep_kernel,
        out_shape=[
            jax.ShapeDtypeStruct((N, DW), I32),
            jax.ShapeDtypeStruct((N, HW), I32),
            jax.ShapeDtypeStruct((N, HW), I32),
            jax.ShapeDtypeStruct((48, HP), F32),
        ],
    )(g_i, w1aT, w1bT, mention_scores, b1p, e48, w1dT)

    sc = [pl.kernel(
        functools.partial(_sc_body, h),
        out_type=[
            jax.ShapeDtypeStruct((H, DW), I32),
            jax.ShapeDtypeStruct((H, DW), I32),
            jax.ShapeDtypeStruct((H, HW), I32),
            jax.ShapeDtypeStruct((H, HW), I32),
        ],
        mesh=plsc.VectorSubcoreMesh(core_axis_name="c", subcore_axis_name="s",
                                    num_cores=NC, num_subcores=NS),
        scratch_types=[
            pltpu.VMEM((PPW,), jnp.int32),
            pltpu.VMEM((PPW,), jnp.int32),
            pltpu.VMEM((3 * C, DW), I32),
            pltpu.VMEM((3 * C, DW), I32),
            pltpu.VMEM((3 * C, HW), I32),
            pltpu.VMEM((3 * C, HW), I32),
        ] + [pltpu.SemaphoreType.DMA] * 8,
    ) for h in range(2)]
    NB2 = P // B
    mid3 = mention_ids.reshape(NB2, 1, B)
    did3 = distance_ids.reshape(NB2, 1, B)
    gid3 = genre_ids.reshape(NB2, 1, B)
    sid3 = speaker_ids.reshape(NB2, 1, B)
    w2T = jnp.pad(W2.T, ((0, pad), (0, pad)))

    row = lambda i: (i, 0)
    blk3 = lambda i: (i, 0, 0)
    full2 = lambda i: (0, 0)

    def make_mlp(h):
        hblk = lambda i: (i + h * NBH, 0, 0)
        return pl.pallas_call(
            _mlp_kernel,
            grid=(NBH,),
            in_specs=[
                pl.BlockSpec((B, DW), row),
                pl.BlockSpec((B, DW), row),
                pl.BlockSpec((B, HW), row),
                pl.BlockSpec((B, HW), row),
                pl.BlockSpec((1, 1, B), hblk),
                pl.BlockSpec((1, 1, B), hblk),
                pl.BlockSpec((1, 1, B), hblk),
                pl.BlockSpec((1, 1, B), hblk),
                pl.BlockSpec((D, HP), full2),
                pl.BlockSpec((48, HP), full2),
                pl.BlockSpec((HP, HP), full2),
                pl.BlockSpec((1, HP), full2),
                pl.BlockSpec((1, HP), full2),
                pl.BlockSpec((1, 1), full2),
                pl.BlockSpec((1, HP), full2),
                pl.BlockSpec((1, HP), full2),
            ],
            out_specs=[
                pl.BlockSpec((1, 1, B), blk3),
                pl.BlockSpec((1, N), full2),
            ],
            out_shape=[
                jax.ShapeDtypeStruct((NBH, 1, B), F32),
                jax.ShapeDtypeStruct((1, N), F32),
            ],
        )

    w1c_hi = w1cT.astype(BF16)
    col = np.arange(HP)
    mh1 = jnp.asarray((col < HID).astype(np.float32)).reshape(1, HP)
    msv = jnp.asarray(((col >= HID) & (col <= HID + 3))
                      .astype(np.float32)).reshape(1, HP)

    es, dens = [], []
    for h in range(2):
        xi, xj, am, aa = sc[h](g3, gm3, ga3, mention_ids, antecedent_ids)
        e3, den = make_mlp(h)(xi, xj, am, aa, mid3, did3, gid3, sid3,
                              w1c_hi, phiT, w2T, b2p, w3row, b3a, mh1, msv)
        es.append(e3)
        dens.append(den)

    ps = []
    for h in range(2):
        hblk = lambda i, h=h: (i + h * NBH, 0, 0)
        p3, eps = pl.pallas_call(
            _probs_kernel,
            grid=(NBH,),
            in_specs=[
                pl.BlockSpec((1, 1, B), blk3),
                pl.BlockSpec((1, 1, B), hblk),
                pl.BlockSpec((1, N), full2),
                pl.BlockSpec((1, N), full2),
            ],
            out_specs=[
                pl.BlockSpec((1, 1, B), blk3),
                pl.BlockSpec((1, N), full2),
            ],
            out_shape=[
                jax.ShapeDtypeStruct((NBH, 1, B), F32),
                jax.ShapeDtypeStruct((1, N), F32),
            ],
        )(es[h], mid3, dens[0], dens[1])
        ps.append(p3)

    return jnp.concatenate([ps[0].reshape(H), ps[1].reshape(H),
                            eps.reshape(N)])
